# baseline (speedup 1.0000x reference)
import functools

import jax
import jax.numpy as jnp
from jax import lax
from jax.experimental import pallas as pl
from jax.experimental.pallas import tpu as pltpu
from jax.experimental.pallas import tpu_sc as plsc

_BATCH = 16384
_NF = 26
_VOCAB = 100001
_EMB = 32
_NNUM = 16

_R = _BATCH * _NF
_NC = 2
_NS = 16
_NW = _NC * _NS
_PER_W = _R // _NW
_CH = 1024
_NCH = _PER_W // _CH
_SUB = 128
_NSUB = _CH // _SUB

_TILE = 1024
_NT = _BATCH // _TILE


def _sc_gather_body(xcat_hbm, table_hbm, out_hbm, idx_v, rows_v, sem):
    wid = lax.axis_index("s") * _NC + lax.axis_index("c")
    base = wid * _PER_W

    def chunk(c, carry):
        ci = base // _CH + c
        start = ci * _CH
        pltpu.sync_copy(xcat_hbm.at[ci], idx_v)
        lane = lax.iota(jnp.int32, 16)
        for j in range(_NSUB):
            def addoff(i, carry2, j=j):
                pos = start + (j * 8 + i) * 16 + lane
                off = lax.rem(pos, _NF) * _VOCAB
                sl = pl.ds(i * 16, 16)
                idx_v[j, sl] = idx_v[j, sl] + off
                return carry2
            lax.fori_loop(0, _SUB // 16, addoff, 0)
        copies = [
            pltpu.async_copy(
                table_hbm.at[idx_v.at[j]],
                rows_v.at[pl.ds(j * _SUB, _SUB)],
                sem,
            )
            for j in range(_NSUB)
        ]
        for cp in copies:
            cp.wait()
        pltpu.sync_copy(rows_v, out_hbm.at[ci])
        return carry

    lax.fori_loop(0, _NCH, chunk, 0)


@functools.cache
def _sc_gather():
    return pl.kernel(
        _sc_gather_body,
        out_type=jax.ShapeDtypeStruct((_R // _CH, _CH, _EMB), jnp.float32),
        mesh=plsc.VectorSubcoreMesh(
            core_axis_name="c", subcore_axis_name="s",
            num_cores=_NC, num_subcores=_NS,
        ),
        scratch_types=[
            pltpu.VMEM((_NSUB, _SUB), jnp.int32),
            pltpu.VMEM((_CH, _EMB), jnp.float32),
            pltpu.SemaphoreType.DMA,
        ],
        compiler_params=pltpu.CompilerParams(use_tc_tiling_on_sc=False),
    )


def _mlp_body(xnum_ref, embs_ref, w1n_ref, w1e_ref, b1_ref, gamma_ref,
              beta_ref, w2_ref, b2_ref, w3_ref, b3_ref, out_ref,
              h1_scr, ssum_scr, ssq_scr):
    p = pl.program_id(0)
    i = pl.program_id(1)

    @pl.when(p == 0)
    def _phase0():
        h1 = (
            jnp.dot(xnum_ref[...], w1n_ref[...],
                    preferred_element_type=jnp.float32)
            + jnp.dot(embs_ref[...], w1e_ref[...],
                      preferred_element_type=jnp.float32)
            + b1_ref[...]
        )
        h1_scr[pl.ds(i * _TILE, _TILE), :] = h1

        @pl.when(i == 0)
        def _init():
            ssum_scr[...] = jnp.zeros_like(ssum_scr)
            ssq_scr[...] = jnp.zeros_like(ssq_scr)

        ssum_scr[0:1, :] += jnp.sum(h1, axis=0, keepdims=True)
        ssq_scr[0:1, :] += jnp.sum(h1 * h1, axis=0, keepdims=True)
        out_ref[...] = jnp.zeros_like(out_ref)

    @pl.when(p == 1)
    def _phase1():
        mu = ssum_scr[0:1, :] * (1.0 / _BATCH)
        var = ssq_scr[0:1, :] * (1.0 / _BATCH) - mu * mu
        rstd = lax.rsqrt(var + 1e-5)
        h1 = h1_scr[pl.ds(i * _TILE, _TILE), :]
        h = jnp.maximum((h1 - mu) * (rstd * gamma_ref[...]) + beta_ref[...],
                        0.0)
        h2 = jnp.maximum(
            jnp.dot(h, w2_ref[...], preferred_element_type=jnp.float32)
            + b2_ref[...], 0.0)
        out_ref[...] = (
            jnp.dot(h2, w3_ref[...], preferred_element_type=jnp.float32)
            + b3_ref[...]
        )


def _mlp(x_num, embs, w1n, w1e, b1, gamma, beta, w2, b2, w3, b3):
    return pl.pallas_call(
        _mlp_body,
        grid=(2, _NT),
        in_specs=[
            pl.BlockSpec((_TILE, _NNUM), lambda p, i: (i * (1 - p), 0)),
            pl.BlockSpec((_TILE, _NF * _EMB), lambda p, i: (i * (1 - p), 0)),
            pl.BlockSpec((_NNUM, 128), lambda p, i: (0, 0)),
            pl.BlockSpec((_NF * _EMB, 128), lambda p, i: (0, 0)),
            pl.BlockSpec((1, 128), lambda p, i: (0, 0)),
            pl.BlockSpec((1, 128), lambda p, i: (0, 0)),
            pl.BlockSpec((1, 128), lambda p, i: (0, 0)),
            pl.BlockSpec((128, 64), lambda p, i: (0, 0)),
            pl.BlockSpec((1, 64), lambda p, i: (0, 0)),
            pl.BlockSpec((64, 1), lambda p, i: (0, 0)),
            pl.BlockSpec((1, 1), lambda p, i: (0, 0)),
        ],
        out_specs=pl.BlockSpec((_TILE, 1), lambda p, i: (i, 0)),
        out_shape=jax.ShapeDtypeStruct((_BATCH, 1), jnp.float32),
        scratch_shapes=[
            pltpu.VMEM((_BATCH, 128), jnp.float32),
            pltpu.VMEM((8, 128), jnp.float32),
            pltpu.VMEM((8, 128), jnp.float32),
        ],
        compiler_params=pltpu.CompilerParams(
            dimension_semantics=("arbitrary", "arbitrary"),
        ),
    )(x_num, embs, w1n, w1e, b1, gamma, beta, w2, b2, w3, b3)


def kernel(x_num, x_cat, emb_tables, W1, b1, gamma, beta, W2, b2, W3, b3):
    table2d = emb_tables.reshape(_NF * _VOCAB, _EMB)
    xcat3d = x_cat.reshape(_R // _CH, _NSUB, _SUB)
    embs = _sc_gather()(xcat3d, table2d)
    embs = embs.reshape(_BATCH, _NF * _EMB)
    out = _mlp(
        x_num, embs,
        W1[:_NNUM], W1[_NNUM:],
        b1.reshape(1, 128), gamma.reshape(1, 128), beta.reshape(1, 128),
        W2, b2.reshape(1, 64), W3, b3.reshape(1, 1),
    )
    return out[:, 0]


# speedup vs baseline: 13.1136x; 13.1136x over previous
import functools

import jax
import jax.numpy as jnp
from jax import lax
from jax.experimental import pallas as pl
from jax.experimental.pallas import tpu as pltpu
from jax.experimental.pallas import tpu_sc as plsc

_BATCH = 16384
_NF = 26
_VOCAB = 100001
_VB = 6272
_NVB = 16
_VPAD = _VB * _NVB
_EMB = 32
_NNUM = 16

_R = _BATCH * _NF
_NC = 2
_NS = 16
_NW = _NC * _NS
_PER_W = _R // _NW
_CH = 512
_NCH = _PER_W // _CH
_SUB = 128
_NSUB = _CH // _SUB

_TILE = 512
_NT = _BATCH // _TILE


def _pack_body(in_ref, out_ref):
    y = jnp.swapaxes(in_ref[0], 0, 1)
    out_ref[:, 0:_EMB] = y
    out_ref[:, _EMB:128] = jnp.zeros((_VB, 128 - _EMB), jnp.float32)


def _pack_table(tT):
    return pl.pallas_call(
        _pack_body,
        grid=(_NF, _NVB),
        in_specs=[pl.BlockSpec((1, _EMB, _VB), lambda f, c: (f, 0, c))],
        out_specs=pl.BlockSpec((_VB, 128), lambda f, c: (f * _NVB + c, 0)),
        out_shape=jax.ShapeDtypeStruct((_NF * _VPAD, 128), jnp.float32),
        compiler_params=pltpu.CompilerParams(
            dimension_semantics=("arbitrary", "arbitrary"),
        ),
    )(tT)


def _sc_gather_body(xcat_hbm, table_hbm, out_hbm, idx_v, rows_v, sem):
    wid = lax.axis_index("s") * _NC + lax.axis_index("c")

    def chunk(c, carry):
        ci = wid * _NCH + c
        start = ci * _CH
        f = start // _BATCH
        b0 = start - f * _BATCH
        off = f * _VPAD
        pltpu.sync_copy(xcat_hbm.at[ci], idx_v)
        for q in range(_NSUB):
            def addoff(i, carry2, q=q):
                sl = pl.ds(i * 16, 16)
                idx_v[q, sl] = idx_v[q, sl] + off
                return carry2
            lax.fori_loop(0, _SUB // 16, addoff, 0)
        copies = [
            pltpu.async_copy(
                table_hbm.at[idx_v.at[q]],
                rows_v.at[pl.ds(q * _SUB, _SUB)],
                sem,
            )
            for q in range(_NSUB)
        ]
        for cp in copies:
            cp.wait()
        pltpu.sync_copy(rows_v, out_hbm.at[f].at[pl.ds(b0, _CH)])
        return carry

    lax.fori_loop(0, _NCH, chunk, 0)


@functools.cache
def _sc_gather():
    return pl.kernel(
        _sc_gather_body,
        out_type=jax.ShapeDtypeStruct((_NF, _BATCH, 128), jnp.float32),
        mesh=plsc.VectorSubcoreMesh(
            core_axis_name="c", subcore_axis_name="s",
            num_cores=_NC, num_subcores=_NS,
        ),
        scratch_types=[
            pltpu.VMEM((_NSUB, _SUB), jnp.int32),
            pltpu.VMEM((_CH, 128), jnp.float32),
            pltpu.SemaphoreType.DMA,
        ],
        compiler_params=pltpu.CompilerParams(use_tc_tiling_on_sc=True),
    )


def _mlp_body(xnum_ref, embs_ref, w1n_ref, w1e_ref, b1_ref, gamma_ref,
              beta_ref, w2_ref, b2_ref, w3_ref, b3_ref, out_ref,
              h1_scr, ssum_scr, ssq_scr):
    p = pl.program_id(0)
    i = pl.program_id(1)

    @pl.when(p == 0)
    def _phase0():
        h1 = jnp.dot(xnum_ref[...], w1n_ref[...],
                     preferred_element_type=jnp.float32) + b1_ref[...]
        for f in range(_NF):
            h1 = h1 + jnp.dot(embs_ref[f], w1e_ref[f],
                              preferred_element_type=jnp.float32)
        h1_scr[pl.ds(i * _TILE, _TILE), :] = h1

        @pl.when(i == 0)
        def _init():
            ssum_scr[...] = jnp.zeros_like(ssum_scr)
            ssq_scr[...] = jnp.zeros_like(ssq_scr)

        ssum_scr[0:1, :] += jnp.sum(h1, axis=0, keepdims=True)
        ssq_scr[0:1, :] += jnp.sum(h1 * h1, axis=0, keepdims=True)
        out_ref[...] = jnp.zeros_like(out_ref)

    @pl.when(p == 1)
    def _phase1():
        mu = ssum_scr[0:1, :] * (1.0 / _BATCH)
        var = ssq_scr[0:1, :] * (1.0 / _BATCH) - mu * mu
        rstd = lax.rsqrt(var + 1e-5)
        h1 = h1_scr[pl.ds(i * _TILE, _TILE), :]
        h = jnp.maximum((h1 - mu) * (rstd * gamma_ref[...]) + beta_ref[...],
                        0.0)
        h2 = jnp.maximum(
            jnp.dot(h, w2_ref[...], preferred_element_type=jnp.float32)
            + b2_ref[...], 0.0)
        out_ref[...] = (
            jnp.dot(h2, w3_ref[...], preferred_element_type=jnp.float32)
            + b3_ref[...]
        )


def _mlp(x_num, embs, w1n, w1e, b1, gamma, beta, w2, b2, w3, b3):
    return pl.pallas_call(
        _mlp_body,
        grid=(2, _NT),
        in_specs=[
            pl.BlockSpec((_TILE, _NNUM), lambda p, i: (i * (1 - p), 0)),
            pl.BlockSpec((_NF, _TILE, 128), lambda p, i: (0, i * (1 - p), 0)),
            pl.BlockSpec((_NNUM, 128), lambda p, i: (0, 0)),
            pl.BlockSpec((_NF, 128, 128), lambda p, i: (0, 0, 0)),
            pl.BlockSpec((1, 128), lambda p, i: (0, 0)),
            pl.BlockSpec((1, 128), lambda p, i: (0, 0)),
            pl.BlockSpec((1, 128), lambda p, i: (0, 0)),
            pl.BlockSpec((128, 64), lambda p, i: (0, 0)),
            pl.BlockSpec((1, 64), lambda p, i: (0, 0)),
            pl.BlockSpec((64, 1), lambda p, i: (0, 0)),
            pl.BlockSpec((1, 1), lambda p, i: (0, 0)),
        ],
        out_specs=pl.BlockSpec((_TILE, 1), lambda p, i: (i, 0)),
        out_shape=jax.ShapeDtypeStruct((_BATCH, 1), jnp.float32),
        scratch_shapes=[
            pltpu.VMEM((_BATCH, 128), jnp.float32),
            pltpu.VMEM((8, 128), jnp.float32),
            pltpu.VMEM((8, 128), jnp.float32),
        ],
        compiler_params=pltpu.CompilerParams(
            dimension_semantics=("arbitrary", "arbitrary"),
        ),
    )(x_num, embs, w1n, w1e, b1, gamma, beta, w2, b2, w3, b3)


def kernel(x_num, x_cat, emb_tables, W1, b1, gamma, beta, W2, b2, W3, b3):
    tP = _pack_table(emb_tables.transpose(0, 2, 1))
    xcatG = x_cat.T.reshape(_R // _CH, _NSUB, _SUB)
    embs = _sc_gather()(xcatG, tP)
    w1e = jnp.pad(W1[_NNUM:].reshape(_NF, _EMB, 128),
                  ((0, 0), (0, 128 - _EMB), (0, 0)))
    out = _mlp(
        x_num, embs,
        W1[:_NNUM], w1e,
        b1.reshape(1, 128), gamma.reshape(1, 128), beta.reshape(1, 128),
        W2, b2.reshape(1, 64), W3, b3.reshape(1, 1),
    )
    return out[:, 0]


# speedup vs baseline: 15.4564x; 1.1787x over previous
import functools

import jax
import jax.numpy as jnp
from jax import lax
from jax.experimental import pallas as pl
from jax.experimental.pallas import tpu as pltpu
from jax.experimental.pallas import tpu_sc as plsc

_BATCH = 16384
_NF = 26
_VOCAB = 100001
_VB = 12544
_NVB = 8
_VPAD = _VB * _NVB
_EMB = 32
_NNUM = 16

_R = _BATCH * _NF
_NC = 2
_NS = 16
_NW = _NC * _NS
_PER_W = _R // _NW
_CH = 256
_NCH = _PER_W // _CH
_SUB = 128
_NSUB = _CH // _SUB

_TILE = 512
_NT = _BATCH // _TILE


def _pack_body(in_ref, out_ref):
    y = jnp.swapaxes(in_ref[0], 0, 1)
    out_ref[:, 0:_EMB] = y
    out_ref[:, _EMB:128] = jnp.zeros((_VB, 128 - _EMB), jnp.float32)


def _pack_table(tT):
    return pl.pallas_call(
        _pack_body,
        grid=(_NF, _NVB),
        in_specs=[pl.BlockSpec((1, _EMB, _VB), lambda f, c: (f, 0, c))],
        out_specs=pl.BlockSpec((_VB, 128), lambda f, c: (f * _NVB + c, 0)),
        out_shape=jax.ShapeDtypeStruct((_NF * _VPAD, 128), jnp.float32),
        compiler_params=pltpu.CompilerParams(
            dimension_semantics=("arbitrary", "arbitrary"),
        ),
    )(tT)


def _sc_gather_body(xcat_hbm, table_hbm, out_hbm,
                    idx_a, idx_b, rows_a, rows_b, sem_a, sem_b):
    wid = lax.axis_index("s") * _NC + lax.axis_index("c")
    base = wid * _NCH

    def prep_fire(ci, idx_v, rows_v, sem):
        off = ((ci * _CH) // _BATCH) * _VPAD
        pltpu.sync_copy(xcat_hbm.at[ci], idx_v)
        for q in range(_NSUB):
            def addoff(i, carry2, q=q):
                sl = pl.ds(i * 16, 16)
                idx_v[q, sl] = idx_v[q, sl] + off
                return carry2
            lax.fori_loop(0, _SUB // 16, addoff, 0)
        for q in range(_NSUB):
            pltpu.async_copy(
                table_hbm.at[idx_v.at[q]],
                rows_v.at[pl.ds(q * _SUB, _SUB)],
                sem,
            )

    def drain_wb(ci, rows_v, sem):
        for q in range(_NSUB):
            pltpu.make_async_copy(
                table_hbm.at[pl.ds(0, _SUB)],
                rows_v.at[pl.ds(q * _SUB, _SUB)],
                sem,
            ).wait()
        start = ci * _CH
        f = start // _BATCH
        b0 = start - f * _BATCH
        pltpu.sync_copy(rows_v, out_hbm.at[f].at[pl.ds(b0, _CH)])

    prep_fire(base, idx_a, rows_a, sem_a)

    def pair(t, carry):
        ca = base + 2 * t
        cb = ca + 1
        prep_fire(cb, idx_b, rows_b, sem_b)
        drain_wb(ca, rows_a, sem_a)

        @pl.when(t < _NCH // 2 - 1)
        def _fire_next():
            prep_fire(ca + 2, idx_a, rows_a, sem_a)

        drain_wb(cb, rows_b, sem_b)
        return carry

    lax.fori_loop(0, _NCH // 2, pair, 0)


@functools.cache
def _sc_gather():
    return pl.kernel(
        _sc_gather_body,
        out_type=jax.ShapeDtypeStruct((_NF, _BATCH, 128), jnp.float32),
        mesh=plsc.VectorSubcoreMesh(
            core_axis_name="c", subcore_axis_name="s",
            num_cores=_NC, num_subcores=_NS,
        ),
        scratch_types=[
            pltpu.VMEM((_NSUB, _SUB), jnp.int32),
            pltpu.VMEM((_NSUB, _SUB), jnp.int32),
            pltpu.VMEM((_CH, 128), jnp.float32),
            pltpu.VMEM((_CH, 128), jnp.float32),
            pltpu.SemaphoreType.DMA,
            pltpu.SemaphoreType.DMA,
        ],
        compiler_params=pltpu.CompilerParams(use_tc_tiling_on_sc=True),
    )


def _mlp_body(xnum_ref, embs_ref, w1n_ref, w1e_ref, b1_ref, gamma_ref,
              beta_ref, w2_ref, b2_ref, w3_ref, b3_ref, out_ref,
              h1_scr, ssum_scr, ssq_scr):
    p = pl.program_id(0)
    i = pl.program_id(1)

    @pl.when(p == 0)
    def _phase0():
        h1 = jnp.dot(xnum_ref[...], w1n_ref[...],
                     preferred_element_type=jnp.float32) + b1_ref[...]
        for f in range(_NF):
            h1 = h1 + jnp.dot(embs_ref[f], w1e_ref[f],
                              preferred_element_type=jnp.float32)
        h1_scr[pl.ds(i * _TILE, _TILE), :] = h1

        @pl.when(i == 0)
        def _init():
            ssum_scr[...] = jnp.zeros_like(ssum_scr)
            ssq_scr[...] = jnp.zeros_like(ssq_scr)

        ssum_scr[0:1, :] += jnp.sum(h1, axis=0, keepdims=True)
        ssq_scr[0:1, :] += jnp.sum(h1 * h1, axis=0, keepdims=True)
        out_ref[...] = jnp.zeros_like(out_ref)

    @pl.when(p == 1)
    def _phase1():
        mu = ssum_scr[0:1, :] * (1.0 / _BATCH)
        var = ssq_scr[0:1, :] * (1.0 / _BATCH) - mu * mu
        rstd = lax.rsqrt(var + 1e-5)
        h1 = h1_scr[pl.ds(i * _TILE, _TILE), :]
        h = jnp.maximum((h1 - mu) * (rstd * gamma_ref[...]) + beta_ref[...],
                        0.0)
        h2 = jnp.maximum(
            jnp.dot(h, w2_ref[...], preferred_element_type=jnp.float32)
            + b2_ref[...], 0.0)
        out_ref[...] = (
            jnp.dot(h2, w3_ref[...], preferred_element_type=jnp.float32)
            + b3_ref[...]
        )


def _mlp(x_num, embs, w1n, w1e, b1, gamma, beta, w2, b2, w3, b3):
    return pl.pallas_call(
        _mlp_body,
        grid=(2, _NT),
        in_specs=[
            pl.BlockSpec((_TILE, _NNUM), lambda p, i: (i * (1 - p), 0)),
            pl.BlockSpec((_NF, _TILE, 128), lambda p, i: (0, i * (1 - p), 0)),
            pl.BlockSpec((_NNUM, 128), lambda p, i: (0, 0)),
            pl.BlockSpec((_NF, 128, 128), lambda p, i: (0, 0, 0)),
            pl.BlockSpec((1, 128), lambda p, i: (0, 0)),
            pl.BlockSpec((1, 128), lambda p, i: (0, 0)),
            pl.BlockSpec((1, 128), lambda p, i: (0, 0)),
            pl.BlockSpec((128, 64), lambda p, i: (0, 0)),
            pl.BlockSpec((1, 64), lambda p, i: (0, 0)),
            pl.BlockSpec((64, 1), lambda p, i: (0, 0)),
            pl.BlockSpec((1, 1), lambda p, i: (0, 0)),
        ],
        out_specs=pl.BlockSpec((_TILE, 1), lambda p, i: (i, 0)),
        out_shape=jax.ShapeDtypeStruct((_BATCH, 1), jnp.float32),
        scratch_shapes=[
            pltpu.VMEM((_BATCH, 128), jnp.float32),
            pltpu.VMEM((8, 128), jnp.float32),
            pltpu.VMEM((8, 128), jnp.float32),
        ],
        compiler_params=pltpu.CompilerParams(
            dimension_semantics=("arbitrary", "arbitrary"),
        ),
    )(x_num, embs, w1n, w1e, b1, gamma, beta, w2, b2, w3, b3)


def kernel(x_num, x_cat, emb_tables, W1, b1, gamma, beta, W2, b2, W3, b3):
    tP = _pack_table(emb_tables.transpose(0, 2, 1))
    xcatG = x_cat.T.reshape(_R // _CH, _NSUB, _SUB)
    embs = _sc_gather()(xcatG, tP)
    w1e = jnp.pad(W1[_NNUM:].reshape(_NF, _EMB, 128),
                  ((0, 0), (0, 128 - _EMB), (0, 0)))
    out = _mlp(
        x_num, embs,
        W1[:_NNUM], w1e,
        b1.reshape(1, 128), gamma.reshape(1, 128), beta.reshape(1, 128),
        W2, b2.reshape(1, 64), W3, b3.reshape(1, 1),
    )
    return out[:, 0]


# speedup vs baseline: 16.2976x; 1.0544x over previous
import functools

import jax
import jax.numpy as jnp
from jax import lax
from jax.experimental import pallas as pl
from jax.experimental.pallas import tpu as pltpu
from jax.experimental.pallas import tpu_sc as plsc

_BATCH = 16384
_NF = 26
_VOCAB = 100001
_VB = 25088
_NVB = 4
_VPAD = _VB * _NVB
_EMB = 32
_NNUM = 16

_R = _BATCH * _NF
_NC = 2
_NS = 16
_NW = _NC * _NS
_PER_W = _R // _NW
_CH = 256
_NCH = _PER_W // _CH
_SUB = 128
_NSUB = _CH // _SUB

_TILE = 1024
_NT = _BATCH // _TILE


def _pack_body(in_ref, out_ref):
    y = jnp.swapaxes(in_ref[0], 0, 1)
    out_ref[:, 0:_EMB] = y
    out_ref[:, _EMB:128] = jnp.zeros((_VB, 128 - _EMB), jnp.float32)


def _pack_table(tT):
    return pl.pallas_call(
        _pack_body,
        grid=(_NF, _NVB),
        in_specs=[pl.BlockSpec((1, _EMB, _VB), lambda f, c: (f, 0, c))],
        out_specs=pl.BlockSpec((_VB, 128), lambda f, c: (f * _NVB + c, 0)),
        out_shape=jax.ShapeDtypeStruct((_NF * _VPAD, 128), jnp.float32),
        compiler_params=pltpu.CompilerParams(
            dimension_semantics=("arbitrary", "arbitrary"),
        ),
    )(tT)


def _sc_gather_body(xcat_hbm, table_hbm, out_hbm,
                    idx_a, idx_b, rows_a, rows_b, sem_a, sem_b):
    wid = lax.axis_index("s") * _NC + lax.axis_index("c")
    base = wid * _NCH

    def prep_fire(ci, idx_v, rows_v, sem):
        off = ((ci * _CH) // _BATCH) * _VPAD
        pltpu.sync_copy(xcat_hbm.at[ci], idx_v)
        for q in range(_NSUB):
            def addoff(i, carry2, q=q):
                sl = pl.ds(i * 16, 16)
                idx_v[q, sl] = idx_v[q, sl] + off
                return carry2
            lax.fori_loop(0, _SUB // 16, addoff, 0)
        for q in range(_NSUB):
            pltpu.async_copy(
                table_hbm.at[idx_v.at[q]],
                rows_v.at[pl.ds(q * _SUB, _SUB)],
                sem,
            )

    def drain_wb(ci, rows_v, sem):
        for q in range(_NSUB):
            pltpu.make_async_copy(
                table_hbm.at[pl.ds(0, _SUB)],
                rows_v.at[pl.ds(q * _SUB, _SUB)],
                sem,
            ).wait()
        start = ci * _CH
        f = start // _BATCH
        b0 = start - f * _BATCH
        pltpu.sync_copy(rows_v, out_hbm.at[f].at[pl.ds(b0, _CH)])

    prep_fire(base, idx_a, rows_a, sem_a)

    def pair(t, carry):
        ca = base + 2 * t
        cb = ca + 1
        prep_fire(cb, idx_b, rows_b, sem_b)
        drain_wb(ca, rows_a, sem_a)

        @pl.when(t < _NCH // 2 - 1)
        def _fire_next():
            prep_fire(ca + 2, idx_a, rows_a, sem_a)

        drain_wb(cb, rows_b, sem_b)
        return carry

    lax.fori_loop(0, _NCH // 2, pair, 0)


@functools.cache
def _sc_gather():
    return pl.kernel(
        _sc_gather_body,
        out_type=jax.ShapeDtypeStruct((_NF, _BATCH, 128), jnp.float32),
        mesh=plsc.VectorSubcoreMesh(
            core_axis_name="c", subcore_axis_name="s",
            num_cores=_NC, num_subcores=_NS,
        ),
        scratch_types=[
            pltpu.VMEM((_NSUB, _SUB), jnp.int32),
            pltpu.VMEM((_NSUB, _SUB), jnp.int32),
            pltpu.VMEM((_CH, 128), jnp.float32),
            pltpu.VMEM((_CH, 128), jnp.float32),
            pltpu.SemaphoreType.DMA,
            pltpu.SemaphoreType.DMA,
        ],
        compiler_params=pltpu.CompilerParams(use_tc_tiling_on_sc=True),
    )


def _mlp_body(xnum_ref, embs_ref, w1n_ref, w1e_ref, b1_ref, gamma_ref,
              beta_ref, w2_ref, b2_ref, w3_ref, b3_ref, out_ref,
              h1_scr, ssum_scr, ssq_scr):
    p = pl.program_id(0)
    i = pl.program_id(1)

    @pl.when(p == 0)
    def _phase0():
        h1 = jnp.dot(xnum_ref[...], w1n_ref[...],
                     preferred_element_type=jnp.float32) + b1_ref[...]
        for f in range(_NF):
            h1 = h1 + jnp.dot(embs_ref[f], w1e_ref[f],
                              preferred_element_type=jnp.float32)
        h1_scr[pl.ds(i * _TILE, _TILE), :] = h1

        @pl.when(i == 0)
        def _init():
            ssum_scr[...] = jnp.zeros_like(ssum_scr)
            ssq_scr[...] = jnp.zeros_like(ssq_scr)

        ssum_scr[0:1, :] += jnp.sum(h1, axis=0, keepdims=True)
        ssq_scr[0:1, :] += jnp.sum(h1 * h1, axis=0, keepdims=True)
        out_ref[...] = jnp.zeros_like(out_ref)

    @pl.when(p == 1)
    def _phase1():
        mu = ssum_scr[0:1, :] * (1.0 / _BATCH)
        var = ssq_scr[0:1, :] * (1.0 / _BATCH) - mu * mu
        rstd = lax.rsqrt(var + 1e-5)
        h1 = h1_scr[pl.ds(i * _TILE, _TILE), :]
        h = jnp.maximum((h1 - mu) * (rstd * gamma_ref[...]) + beta_ref[...],
                        0.0)
        h2 = jnp.maximum(
            jnp.dot(h, w2_ref[...], preferred_element_type=jnp.float32)
            + b2_ref[...], 0.0)
        out_ref[...] = (
            jnp.dot(h2, w3_ref[...], preferred_element_type=jnp.float32)
            + b3_ref[...]
        )


def _mlp(x_num, embs, w1n, w1e, b1, gamma, beta, w2, b2, w3, b3):
    return pl.pallas_call(
        _mlp_body,
        grid=(2, _NT),
        in_specs=[
            pl.BlockSpec((_TILE, _NNUM), lambda p, i: (i * (1 - p), 0)),
            pl.BlockSpec((_NF, _TILE, 128), lambda p, i: (0, i * (1 - p), 0)),
            pl.BlockSpec((_NNUM, 128), lambda p, i: (0, 0)),
            pl.BlockSpec((_NF, 128, 128), lambda p, i: (0, 0, 0)),
            pl.BlockSpec((1, 128), lambda p, i: (0, 0)),
            pl.BlockSpec((1, 128), lambda p, i: (0, 0)),
            pl.BlockSpec((1, 128), lambda p, i: (0, 0)),
            pl.BlockSpec((128, 64), lambda p, i: (0, 0)),
            pl.BlockSpec((1, 64), lambda p, i: (0, 0)),
            pl.BlockSpec((64, 1), lambda p, i: (0, 0)),
            pl.BlockSpec((1, 1), lambda p, i: (0, 0)),
        ],
        out_specs=pl.BlockSpec((_TILE, 1), lambda p, i: (i, 0)),
        out_shape=jax.ShapeDtypeStruct((_BATCH, 1), jnp.float32),
        scratch_shapes=[
            pltpu.VMEM((_BATCH, 128), jnp.float32),
            pltpu.VMEM((8, 128), jnp.float32),
            pltpu.VMEM((8, 128), jnp.float32),
        ],
        compiler_params=pltpu.CompilerParams(
            dimension_semantics=("arbitrary", "arbitrary"),
        ),
    )(x_num, embs, w1n, w1e, b1, gamma, beta, w2, b2, w3, b3)


def kernel(x_num, x_cat, emb_tables, W1, b1, gamma, beta, W2, b2, W3, b3):
    tP = _pack_table(emb_tables.transpose(0, 2, 1))
    xcatG = x_cat.T.reshape(_R // _CH, _NSUB, _SUB)
    embs = _sc_gather()(xcatG, tP)
    w1e = jnp.pad(W1[_NNUM:].reshape(_NF, _EMB, 128),
                  ((0, 0), (0, 128 - _EMB), (0, 0)))
    out = _mlp(
        x_num, embs,
        W1[:_NNUM], w1e,
        b1.reshape(1, 128), gamma.reshape(1, 128), beta.reshape(1, 128),
        W2, b2.reshape(1, 64), W3, b3.reshape(1, 1),
    )
    return out[:, 0]
